# baseline (speedup 1.0000x reference)
import jax
import jax.numpy as jnp
from jax.experimental import pallas as pl

_SEQ_BLK = 512


def _pe_body(x_ref, pe_ref, aid_ref, pid_ref, anat_ref, phase_ref, out_ref):
    x = x_ref[...]
    pe = pe_ref[...]
    aid = aid_ref[0]
    pid = pid_ref[0]

    n_anat = anat_ref.shape[0]
    n_phase = phase_ref.shape[0]
    sb = x.shape[0]

    anat_t = jnp.concatenate([anat_ref[...]] * 4, axis=1)
    phase_t = jnp.concatenate([phase_ref[...]] * 4, axis=1)

    a_lane = jax.lax.broadcasted_iota(jnp.int32, (sb, n_anat), 1)
    p_lane = jax.lax.broadcasted_iota(jnp.int32, (sb, n_phase), 1)
    oh_a = (aid == a_lane).astype(jnp.float32)
    oh_p = (pid == p_lane).astype(jnp.float32)

    enc = jax.lax.dot(oh_a, anat_t, precision=jax.lax.Precision.HIGHEST)
    enc = enc + jax.lax.dot(oh_p, phase_t, precision=jax.lax.Precision.HIGHEST)
    out_ref[...] = x + pe + enc


def kernel(x, anatomical_ids, phase_ids, pe, anat_table, phase_table):
    seq_len, batch, d_model = x.shape
    sblk = min(_SEQ_BLK, seq_len)
    n_sblk = seq_len // sblk

    x2 = x.reshape(seq_len, batch * d_model)
    pe2 = pe.reshape(pe.shape[0], d_model)
    aid = anatomical_ids.astype(jnp.int32).T.reshape(batch, seq_len, 1)
    pid = phase_ids.astype(jnp.int32).T.reshape(batch, seq_len, 1)

    out2 = pl.pallas_call(
        _pe_body,
        grid=(n_sblk, batch),
        in_specs=[
            pl.BlockSpec((sblk, d_model), lambda i, b: (i, b)),
            pl.BlockSpec((sblk, d_model), lambda i, b: (i, 0)),
            pl.BlockSpec((1, sblk, 1), lambda i, b: (b, i, 0)),
            pl.BlockSpec((1, sblk, 1), lambda i, b: (b, i, 0)),
            pl.BlockSpec(anat_table.shape, lambda i, b: (0, 0)),
            pl.BlockSpec(phase_table.shape, lambda i, b: (0, 0)),
        ],
        out_specs=pl.BlockSpec((sblk, d_model), lambda i, b: (i, b)),
        out_shape=jax.ShapeDtypeStruct((seq_len, batch * d_model), x.dtype),
    )(x2, pe2, aid, pid, anat_table, phase_table)
    return out2.reshape(seq_len, batch, d_model)


# speedup vs baseline: 2.2619x; 2.2619x over previous
import jax
import jax.numpy as jnp
from jax.experimental import pallas as pl

_SEQ_BLK = 512


def _pe_body(x_ref, pe_ref, aid_ref, pid_ref, anat_ref, phase_ref, out_ref):
    pe = pe_ref[:, 0, :]
    batch = x_ref.shape[1]
    sb = x_ref.shape[0]
    n_anat = anat_ref.shape[0]
    n_phase = phase_ref.shape[0]

    anat_t = jnp.concatenate([anat_ref[...]] * 4, axis=1)
    phase_t = jnp.concatenate([phase_ref[...]] * 4, axis=1)
    a_lane = jax.lax.broadcasted_iota(jnp.int32, (sb, n_anat), 1)
    p_lane = jax.lax.broadcasted_iota(jnp.int32, (sb, n_phase), 1)

    for b in range(batch):
        aid = aid_ref[:, b:b + 1]
        pid = pid_ref[:, b:b + 1]
        oh_a = (aid == a_lane).astype(jnp.float32)
        oh_p = (pid == p_lane).astype(jnp.float32)
        enc = jax.lax.dot(oh_a, anat_t,
                          precision=jax.lax.Precision.HIGHEST)
        enc = enc + jax.lax.dot(oh_p, phase_t,
                                precision=jax.lax.Precision.HIGHEST)
        out_ref[:, b, :] = x_ref[:, b, :] + pe + enc


def kernel(x, anatomical_ids, phase_ids, pe, anat_table, phase_table):
    seq_len, batch, d_model = x.shape
    sblk = min(_SEQ_BLK, seq_len)
    n_sblk = seq_len // sblk

    aid = anatomical_ids.astype(jnp.int32)
    pid = phase_ids.astype(jnp.int32)

    return pl.pallas_call(
        _pe_body,
        grid=(n_sblk,),
        in_specs=[
            pl.BlockSpec((sblk, batch, d_model), lambda i: (i, 0, 0)),
            pl.BlockSpec((sblk, 1, d_model), lambda i: (i, 0, 0)),
            pl.BlockSpec((sblk, batch), lambda i: (i, 0)),
            pl.BlockSpec((sblk, batch), lambda i: (i, 0)),
            pl.BlockSpec(anat_table.shape, lambda i: (0, 0)),
            pl.BlockSpec(phase_table.shape, lambda i: (0, 0)),
        ],
        out_specs=pl.BlockSpec((sblk, batch, d_model), lambda i: (i, 0, 0)),
        out_shape=jax.ShapeDtypeStruct((seq_len, batch, d_model), x.dtype),
    )(x, pe, aid, pid, anat_table, phase_table)


# speedup vs baseline: 3.9170x; 1.7317x over previous
import jax
import jax.numpy as jnp
from jax.experimental import pallas as pl

_SEQ_BLK = 512


def _pe_body(x_ref, pe_ref, aid_ref, pid_ref, anat_ref, phase_ref, out_ref):
    sb, batch, d_model = x_ref.shape
    n_anat = anat_ref.shape[0]
    n_phase = phase_ref.shape[0]
    n_comb = n_anat * n_phase

    anat_t = jnp.concatenate([anat_ref[...]] * 4, axis=1)
    phase_t = jnp.concatenate([phase_ref[...]] * 4, axis=1)

    c_row_a = jax.lax.broadcasted_iota(jnp.int32, (n_comb, n_anat), 0)
    c_lane_a = jax.lax.broadcasted_iota(jnp.int32, (n_comb, n_anat), 1)
    c_row_p = jax.lax.broadcasted_iota(jnp.int32, (n_comb, n_phase), 0)
    c_lane_p = jax.lax.broadcasted_iota(jnp.int32, (n_comb, n_phase), 1)
    e_a = (c_row_a // n_phase == c_lane_a).astype(jnp.float32)
    e_p = (c_row_p % n_phase == c_lane_p).astype(jnp.float32)
    comb = jax.lax.dot(e_a, anat_t, precision=jax.lax.Precision.HIGHEST)
    comb = comb + jax.lax.dot(e_p, phase_t, precision=jax.lax.Precision.HIGHEST)

    cid = aid_ref[...] * n_phase + pid_ref[...]
    lane = jax.lax.broadcasted_iota(jnp.int32, (sb, batch, n_comb), 2)
    oh = (cid[:, :, None] == lane).astype(jnp.float32)
    enc = jax.lax.dot_general(
        oh, comb, (((2,), (0,)), ((), ())),
        precision=jax.lax.Precision.HIGHEST)
    out_ref[...] = x_ref[...] + pe_ref[...] + enc


def kernel(x, anatomical_ids, phase_ids, pe, anat_table, phase_table):
    seq_len, batch, d_model = x.shape
    sblk = min(_SEQ_BLK, seq_len)
    n_sblk = seq_len // sblk

    aid = anatomical_ids.astype(jnp.int32)
    pid = phase_ids.astype(jnp.int32)

    return pl.pallas_call(
        _pe_body,
        grid=(n_sblk,),
        in_specs=[
            pl.BlockSpec((sblk, batch, d_model), lambda i: (i, 0, 0)),
            pl.BlockSpec((sblk, 1, d_model), lambda i: (i, 0, 0)),
            pl.BlockSpec((sblk, batch), lambda i: (i, 0)),
            pl.BlockSpec((sblk, batch), lambda i: (i, 0)),
            pl.BlockSpec(anat_table.shape, lambda i: (0, 0)),
            pl.BlockSpec(phase_table.shape, lambda i: (0, 0)),
        ],
        out_specs=pl.BlockSpec((sblk, batch, d_model), lambda i: (i, 0, 0)),
        out_shape=jax.ShapeDtypeStruct((seq_len, batch, d_model), x.dtype),
    )(x, pe, aid, pid, anat_table, phase_table)


# speedup vs baseline: 4.0171x; 1.0255x over previous
import jax
import jax.numpy as jnp
from jax.experimental import pallas as pl

_SEQ_BLK = 256


def _pe_body(x_ref, pe_ref, aid_ref, pid_ref, anat_ref, phase_ref, out_ref):
    sb, batch, d_model = x_ref.shape
    n_anat = anat_ref.shape[0]
    n_phase = phase_ref.shape[0]
    n_comb = n_anat * n_phase

    anat_t = jnp.concatenate([anat_ref[...]] * 4, axis=1)
    phase_t = jnp.concatenate([phase_ref[...]] * 4, axis=1)

    c_row_a = jax.lax.broadcasted_iota(jnp.int32, (n_comb, n_anat), 0)
    c_lane_a = jax.lax.broadcasted_iota(jnp.int32, (n_comb, n_anat), 1)
    c_row_p = jax.lax.broadcasted_iota(jnp.int32, (n_comb, n_phase), 0)
    c_lane_p = jax.lax.broadcasted_iota(jnp.int32, (n_comb, n_phase), 1)
    e_a = (c_row_a // n_phase == c_lane_a).astype(jnp.float32)
    e_p = (c_row_p % n_phase == c_lane_p).astype(jnp.float32)
    comb = jax.lax.dot(e_a, anat_t, precision=jax.lax.Precision.HIGHEST)
    comb = comb + jax.lax.dot(e_p, phase_t, precision=jax.lax.Precision.HIGHEST)

    cid = aid_ref[...] * n_phase + pid_ref[...]
    lane = jax.lax.broadcasted_iota(jnp.int32, (sb, batch, n_comb), 2)
    oh = (cid[:, :, None] == lane).astype(jnp.float32)
    enc = jax.lax.dot_general(
        oh, comb, (((2,), (0,)), ((), ())),
        precision=jax.lax.Precision.HIGHEST)
    out_ref[...] = x_ref[...] + pe_ref[...] + enc


def kernel(x, anatomical_ids, phase_ids, pe, anat_table, phase_table):
    seq_len, batch, d_model = x.shape
    sblk = min(_SEQ_BLK, seq_len)
    n_sblk = seq_len // sblk

    aid = anatomical_ids.astype(jnp.int32)
    pid = phase_ids.astype(jnp.int32)

    return pl.pallas_call(
        _pe_body,
        grid=(n_sblk,),
        in_specs=[
            pl.BlockSpec((sblk, batch, d_model), lambda i: (i, 0, 0)),
            pl.BlockSpec((sblk, 1, d_model), lambda i: (i, 0, 0)),
            pl.BlockSpec((sblk, batch), lambda i: (i, 0)),
            pl.BlockSpec((sblk, batch), lambda i: (i, 0)),
            pl.BlockSpec(anat_table.shape, lambda i: (0, 0)),
            pl.BlockSpec(phase_table.shape, lambda i: (0, 0)),
        ],
        out_specs=pl.BlockSpec((sblk, batch, d_model), lambda i: (i, 0, 0)),
        out_shape=jax.ShapeDtypeStruct((seq_len, batch, d_model), x.dtype),
    )(x, pe, aid, pid, anat_table, phase_table)
